# baseline (speedup 1.0000x reference)
import functools

import jax
import jax.numpy as jnp
from jax import lax
from jax.experimental import pallas as pl
from jax.experimental.pallas import tpu as pltpu
from jax.experimental.pallas import tpu_sc as plsc

N = 10000
E = 320000
D1 = 128
D2 = 64
NPAD = 10240
NC = 2
NS = 16
NW = NC * NS
CH = 128
K = 80
IB = 16
NB = K // IB
E_PAD = NW * K * CH
ROWS_PER_TILE = NPAD // NS

_MESH = plsc.VectorSubcoreMesh(
    core_axis_name="c", subcore_axis_name="s", num_cores=NC, num_subcores=NS)


@functools.partial(
    pl.kernel,
    out_type=jax.ShapeDtypeStruct((NC, NPAD, 16), jnp.float32),
    mesh=_MESH,
    scratch_types=[
        pltpu.VMEM((K, CH), jnp.int32),
        pltpu.VMEM((CH, 16), jnp.float32),
        pltpu.VMEM((16, 16), jnp.float32),
        pltpu.VMEM((160, 16), jnp.float32),
        pltpu.VMEM_SHARED((NPAD, 16), jnp.float32),
        pltpu.SemaphoreType.DMA,
    ],
)
def _deg_sc(col_hbm, out_hbm, colv, onesv, zv, obuf, acc, sem):
    cid = lax.axis_index("c")
    sid = lax.axis_index("s")
    wid = cid * NS + sid
    row0 = sid * ROWS_PER_TILE

    def _fill_ones(i, _):
        onesv[i, :] = jnp.full((16,), 1.0, jnp.float32)
        return _
    lax.fori_loop(0, CH, _fill_ones, 0)

    def _fill_z(i, _):
        zv[i, :] = jnp.zeros((16,), jnp.float32)
        return _
    lax.fori_loop(0, 16, _fill_z, 0)

    def _zero_acc(j, _):
        pltpu.sync_copy(zv, acc.at[pl.ds(row0 + 16 * j, 16)])
        return _
    lax.fori_loop(0, ROWS_PER_TILE // 16, _zero_acc, 0)

    pltpu.sync_copy(col_hbm.at[wid], colv)
    plsc.subcore_barrier()

    def _scatter(j, _):
        pltpu.sync_copy(onesv, acc.at[colv.at[j]], add=True)
        return _
    lax.fori_loop(0, K, _scatter, 0)
    plsc.subcore_barrier()

    def _writeout(j, _):
        r = row0 + 160 * j
        pltpu.sync_copy(acc.at[pl.ds(r, 160)], obuf)
        pltpu.sync_copy(obuf, out_hbm.at[cid, pl.ds(r, 160)])
        return _
    lax.fori_loop(0, ROWS_PER_TILE // 160, _writeout, 0)


def _make_push(D):
    @functools.partial(
        pl.kernel,
        out_type=jax.ShapeDtypeStruct((NC, NPAD, D), jnp.float32),
        mesh=_MESH,
        scratch_types=[
            pltpu.VMEM((IB, CH), jnp.int32),
            pltpu.VMEM((IB, CH), jnp.int32),
            pltpu.VMEM((IB, CH), jnp.int32),
            pltpu.VMEM((IB, CH), jnp.int32),
            pltpu.VMEM((CH, D), jnp.float32),
            pltpu.VMEM((CH, D), jnp.float32),
            pltpu.VMEM((16, D), jnp.float32),
            pltpu.VMEM_SHARED((NPAD, D), jnp.float32),
            pltpu.SemaphoreType.DMA,
            pltpu.SemaphoreType.DMA,
            pltpu.SemaphoreType.DMA,
        ],
    )
    def _push(g_hbm, row_hbm, col_hbm, out_hbm, rv0, rv1, cv0, cv1,
              bufa, bufb, zv, acc, sema, semb, isem):
        cid = lax.axis_index("c")
        sid = lax.axis_index("s")
        wid = cid * NS + sid
        row0 = sid * ROWS_PER_TILE

        def _fill_z(i, _):
            for t in range(D // 16):
                zv[i, pl.ds(16 * t, 16)] = jnp.zeros((16,), jnp.float32)
            return _
        lax.fori_loop(0, 16, _fill_z, 0)

        def _zero_acc(j, _):
            pltpu.sync_copy(zv, acc.at[pl.ds(row0 + 16 * j, 16)])
            return _
        lax.fori_loop(0, ROWS_PER_TILE // 16, _zero_acc, 0)

        pltpu.sync_copy(row_hbm.at[wid, pl.ds(0, IB)], rv0)
        pltpu.sync_copy(col_hbm.at[wid, pl.ds(0, IB)], cv0)
        plsc.subcore_barrier()

        for blk in range(NB):
            rv, cv = (rv0, cv0) if blk % 2 == 0 else (rv1, cv1)
            nrv, ncv = (rv1, cv1) if blk % 2 == 0 else (rv0, cv0)
            if blk + 1 < NB:
                pltpu.async_copy(row_hbm.at[wid, pl.ds((blk + 1) * IB, IB)],
                                 nrv, isem)
                pltpu.async_copy(col_hbm.at[wid, pl.ds((blk + 1) * IB, IB)],
                                 ncv, isem)

            def _pair(t, _, rv=rv, cv=cv):
                a = 2 * t
                b = a + 1
                da = pltpu.async_copy(g_hbm.at[rv.at[a]], bufa, sema)
                db = pltpu.async_copy(g_hbm.at[rv.at[b]], bufb, semb)
                da.wait()
                pltpu.sync_copy(bufa, acc.at[cv.at[a]], add=True)
                db.wait()
                pltpu.sync_copy(bufb, acc.at[cv.at[b]], add=True)
                return _
            lax.fori_loop(0, IB // 2, _pair, 0)

            if blk + 1 < NB:
                pltpu.make_async_copy(
                    row_hbm.at[wid, pl.ds((blk + 1) * IB, IB)], nrv,
                    isem).wait()
                pltpu.make_async_copy(
                    col_hbm.at[wid, pl.ds((blk + 1) * IB, IB)], ncv,
                    isem).wait()
        plsc.subcore_barrier()

        def _writeout(j, _):
            r = row0 + CH * j
            pltpu.sync_copy(acc.at[pl.ds(r, CH)], bufa)
            pltpu.sync_copy(bufa, out_hbm.at[cid, pl.ds(r, CH)])
            return _
        lax.fori_loop(0, ROWS_PER_TILE // CH, _writeout, 0)

    return _push


_push128 = _make_push(D1)


_BM = 2048


def _mm_body(x_ref, w_ref, o_ref):
    o_ref[:] = jnp.dot(x_ref[:], w_ref[:], preferred_element_type=jnp.float32)


def _mm1(x_pad, W1):
    return pl.pallas_call(
        _mm_body,
        grid=(NPAD // _BM,),
        in_specs=[
            pl.BlockSpec((_BM, D1), lambda i: (i, 0)),
            pl.BlockSpec((D1, D1), lambda i: (0, 0)),
        ],
        out_specs=pl.BlockSpec((_BM, D1), lambda i: (i, 0)),
        out_shape=jax.ShapeDtypeStruct((NPAD, D1), jnp.float32),
    )(x_pad, W1)


def _scale_body(p_ref, xw_ref, dinvb_ref, g_ref):
    deg = p_ref[0, :, :] + p_ref[1, :, :] + 1.0
    dv = lax.rsqrt(deg)
    dinvb = dv[:, 0:1] * jnp.ones((1, D1), jnp.float32)
    dinvb_ref[:] = dinvb
    g_ref[:] = xw_ref[:] * dinvb


def _scale(deg_parts, xw1):
    return pl.pallas_call(
        _scale_body,
        grid=(NPAD // _BM,),
        in_specs=[
            pl.BlockSpec((NC, _BM, 16), lambda i: (0, i, 0)),
            pl.BlockSpec((_BM, D1), lambda i: (i, 0)),
        ],
        out_specs=[
            pl.BlockSpec((_BM, D1), lambda i: (i, 0)),
            pl.BlockSpec((_BM, D1), lambda i: (i, 0)),
        ],
        out_shape=[
            jax.ShapeDtypeStruct((NPAD, D1), jnp.float32),
            jax.ShapeDtypeStruct((NPAD, D1), jnp.float32),
        ],
    )(deg_parts, xw1)


def _mm2_body(s1a_ref, s1b_ref, g1_ref, dinvb_ref, b1_ref, w2_ref, g2_ref):
    pre = (s1a_ref[:] + s1b_ref[:] + g1_ref[:]) * dinvb_ref[:] + b1_ref[:]
    h1 = jnp.maximum(pre, 0.0)
    rid = pl.program_id(0) * _BM + lax.broadcasted_iota(jnp.int32, (_BM, 1), 0)
    h1 = jnp.where(rid < N, h1, 0.0)
    g2 = jnp.dot(h1, w2_ref[:], preferred_element_type=jnp.float32)
    g2_ref[:] = g2 * dinvb_ref[:]


def _mm2(s1a, s1b, g1, dinvb, b1r, W2p):
    return pl.pallas_call(
        _mm2_body,
        grid=(NPAD // _BM,),
        in_specs=[
            pl.BlockSpec((_BM, D1), lambda i: (i, 0)),
            pl.BlockSpec((_BM, D1), lambda i: (i, 0)),
            pl.BlockSpec((_BM, D1), lambda i: (i, 0)),
            pl.BlockSpec((_BM, D1), lambda i: (i, 0)),
            pl.BlockSpec((1, D1), lambda i: (0, 0)),
            pl.BlockSpec((D1, D1), lambda i: (0, 0)),
        ],
        out_specs=pl.BlockSpec((_BM, D1), lambda i: (i, 0)),
        out_shape=jax.ShapeDtypeStruct((NPAD, D1), jnp.float32),
    )(s1a, s1b, g1, dinvb, b1r, W2p)


_BM3 = 2000


def _mm3_body(s2a_ref, s2b_ref, g2_ref, dinvb_ref, b2_ref, o_ref):
    s = (s2a_ref[:] + s2b_ref[:] + g2_ref[:]) * dinvb_ref[:]
    o_ref[:] = s[:, :D2] + b2_ref[:]


def _mm3(s2a, s2b, g2, dinvb, b2r):
    return pl.pallas_call(
        _mm3_body,
        grid=(N // _BM3,),
        in_specs=[
            pl.BlockSpec((_BM3, D1), lambda i: (i, 0)),
            pl.BlockSpec((_BM3, D1), lambda i: (i, 0)),
            pl.BlockSpec((_BM3, D1), lambda i: (i, 0)),
            pl.BlockSpec((_BM3, D1), lambda i: (i, 0)),
            pl.BlockSpec((1, D2), lambda i: (0, 0)),
        ],
        out_specs=pl.BlockSpec((_BM3, D2), lambda i: (i, 0)),
        out_shape=jax.ShapeDtypeStruct((N, D2), jnp.float32),
    )(s2a, s2b, g2, dinvb, b2r)


def kernel(x, edge_index, W1, b1, W2, b2):
    ei = edge_index.astype(jnp.int32)
    pad = jnp.full((E_PAD - E,), N, jnp.int32)
    rowp = jnp.concatenate([ei[0], pad]).reshape(NW, K, CH)
    colp = jnp.concatenate([ei[1], pad]).reshape(NW, K, CH)
    x_pad = jnp.pad(x, ((0, NPAD - N), (0, 0)))

    W2p = jnp.pad(W2, ((0, 0), (0, D1 - D2)))

    deg_parts = _deg_sc(colp)
    xw1 = _mm1(x_pad, W1)
    dinvb, g1 = _scale(deg_parts, xw1)
    s1 = _push128(g1, rowp, colp)
    g2 = _mm2(s1[0], s1[1], g1, dinvb, b1.reshape(1, D1), W2p)
    s2 = _push128(g2, rowp, colp)
    return _mm3(s2[0], s2[1], g2, dinvb, b2.reshape(1, D2))


# speedup vs baseline: 1.4555x; 1.4555x over previous
import functools

import jax
import jax.numpy as jnp
from jax import lax
from jax.experimental import pallas as pl
from jax.experimental.pallas import tpu as pltpu
from jax.experimental.pallas import tpu_sc as plsc

N = 10000
E = 320000
D1 = 128
D2 = 64
NPAD = 10240
NC = 2
NS = 16
NW = NC * NS
CH = 128
K = -(-E // (NW * CH))
E_PAD = NW * K * CH
ZR = 16
ROWS_PER_TILE = NPAD // NS

_MESH = plsc.VectorSubcoreMesh(
    core_axis_name="c", subcore_axis_name="s", num_cores=NC, num_subcores=NS)


@functools.partial(
    pl.kernel,
    out_type=jax.ShapeDtypeStruct((NC, NPAD, 16), jnp.float32),
    mesh=_MESH,
    scratch_types=[
        pltpu.VMEM((K, CH), jnp.int32),
        pltpu.VMEM((CH, 16), jnp.float32),
        pltpu.VMEM((16, 16), jnp.float32),
        pltpu.VMEM((160, 16), jnp.float32),
        pltpu.VMEM_SHARED((NPAD, 16), jnp.float32),
        pltpu.SemaphoreType.DMA,
    ],
)
def _deg_sc(col_hbm, out_hbm, colv, onesv, zv, obuf, acc, sem):
    cid = lax.axis_index("c")
    sid = lax.axis_index("s")
    wid = cid * NS + sid
    row0 = sid * ROWS_PER_TILE

    def _fill_ones(i, _):
        onesv[i, :] = jnp.full((16,), 1.0, jnp.float32)
        return _
    lax.fori_loop(0, CH, _fill_ones, 0)

    def _fill_z(i, _):
        zv[i, :] = jnp.zeros((16,), jnp.float32)
        return _
    lax.fori_loop(0, 16, _fill_z, 0)

    def _zero_acc(j, _):
        pltpu.sync_copy(zv, acc.at[pl.ds(row0 + 16 * j, 16)])
        return _
    lax.fori_loop(0, ROWS_PER_TILE // 16, _zero_acc, 0)

    pltpu.sync_copy(col_hbm.at[wid], colv)
    plsc.subcore_barrier()

    def _scatter(j, _):
        pltpu.sync_copy(onesv, acc.at[colv.at[j]], add=True)
        return _
    lax.fori_loop(0, K, _scatter, 0)
    plsc.subcore_barrier()

    def _writeout(j, _):
        r = row0 + 160 * j
        pltpu.sync_copy(acc.at[pl.ds(r, 160)], obuf)
        pltpu.sync_copy(obuf, out_hbm.at[cid, pl.ds(r, 160)])
        return _
    lax.fori_loop(0, ROWS_PER_TILE // 160, _writeout, 0)


def _make_push(D):
    @functools.partial(
        pl.kernel,
        out_type=jax.ShapeDtypeStruct((NC, NPAD, D), jnp.float32),
        mesh=_MESH,
        scratch_types=[
            pltpu.VMEM((K, CH), jnp.int32),
            pltpu.VMEM((K, CH), jnp.int32),
            pltpu.VMEM((CH, D), jnp.float32),
            pltpu.VMEM((ZR, D), jnp.float32),
            pltpu.VMEM_SHARED((NPAD, D), jnp.float32),
            pltpu.SemaphoreType.DMA,
        ],
    )
    def _push(g_hbm, row_hbm, col_hbm, out_hbm, rowv, colv, rbuf, zv, acc,
              sem):
        cid = lax.axis_index("c")
        sid = lax.axis_index("s")
        wid = cid * NS + sid
        row0 = sid * ROWS_PER_TILE

        def _fill_z(i, _):
            for t in range(D // 16):
                zv[i, pl.ds(16 * t, 16)] = jnp.zeros((16,), jnp.float32)
            return _
        lax.fori_loop(0, ZR, _fill_z, 0)

        def _zero_acc(j, _):
            pltpu.sync_copy(zv, acc.at[pl.ds(row0 + ZR * j, ZR)])
            return _
        lax.fori_loop(0, ROWS_PER_TILE // ZR, _zero_acc, 0)

        pltpu.sync_copy(row_hbm.at[wid], rowv)
        pltpu.sync_copy(col_hbm.at[wid], colv)
        plsc.subcore_barrier()

        def _edge_chunk(j, _):
            pltpu.async_copy(g_hbm.at[rowv.at[j]], rbuf, sem).wait()
            pltpu.sync_copy(rbuf, acc.at[colv.at[j]], add=True)
            return _
        lax.fori_loop(0, K, _edge_chunk, 0)
        plsc.subcore_barrier()

        def _writeout(j, _):
            r = row0 + CH * j
            pltpu.sync_copy(acc.at[pl.ds(r, CH)], rbuf)
            pltpu.sync_copy(rbuf, out_hbm.at[cid, pl.ds(r, CH)])
            return _
        lax.fori_loop(0, ROWS_PER_TILE // CH, _writeout, 0)

    return _push


_push128 = _make_push(D1)


_BM = 2048


def _mm_body(x_ref, w_ref, o_ref):
    o_ref[:] = jnp.dot(x_ref[:], w_ref[:], preferred_element_type=jnp.float32)


def _mm1(x_pad, W1):
    return pl.pallas_call(
        _mm_body,
        grid=(NPAD // _BM,),
        in_specs=[
            pl.BlockSpec((_BM, D1), lambda i: (i, 0)),
            pl.BlockSpec((D1, D1), lambda i: (0, 0)),
        ],
        out_specs=pl.BlockSpec((_BM, D1), lambda i: (i, 0)),
        out_shape=jax.ShapeDtypeStruct((NPAD, D1), jnp.float32),
    )(x_pad, W1)


def _scale_body(p_ref, xw_ref, dinvb_ref, g_ref):
    deg = p_ref[0, :, :] + p_ref[1, :, :] + 1.0
    dv = lax.rsqrt(deg)
    dinvb = dv[:, 0:1] * jnp.ones((1, D1), jnp.float32)
    dinvb_ref[:] = dinvb
    g_ref[:] = xw_ref[:] * dinvb


def _scale(deg_parts, xw1):
    return pl.pallas_call(
        _scale_body,
        grid=(NPAD // _BM,),
        in_specs=[
            pl.BlockSpec((NC, _BM, 16), lambda i: (0, i, 0)),
            pl.BlockSpec((_BM, D1), lambda i: (i, 0)),
        ],
        out_specs=[
            pl.BlockSpec((_BM, D1), lambda i: (i, 0)),
            pl.BlockSpec((_BM, D1), lambda i: (i, 0)),
        ],
        out_shape=[
            jax.ShapeDtypeStruct((NPAD, D1), jnp.float32),
            jax.ShapeDtypeStruct((NPAD, D1), jnp.float32),
        ],
    )(deg_parts, xw1)


def _mm2_body(s1a_ref, s1b_ref, g1_ref, dinvb_ref, b1_ref, w2_ref, g2_ref):
    pre = (s1a_ref[:] + s1b_ref[:] + g1_ref[:]) * dinvb_ref[:] + b1_ref[:]
    h1 = jnp.maximum(pre, 0.0)
    rid = pl.program_id(0) * _BM + lax.broadcasted_iota(jnp.int32, (_BM, 1), 0)
    h1 = jnp.where(rid < N, h1, 0.0)
    g2 = jnp.dot(h1, w2_ref[:], preferred_element_type=jnp.float32)
    g2_ref[:] = g2 * dinvb_ref[:]


def _mm2(s1a, s1b, g1, dinvb, b1r, W2p):
    return pl.pallas_call(
        _mm2_body,
        grid=(NPAD // _BM,),
        in_specs=[
            pl.BlockSpec((_BM, D1), lambda i: (i, 0)),
            pl.BlockSpec((_BM, D1), lambda i: (i, 0)),
            pl.BlockSpec((_BM, D1), lambda i: (i, 0)),
            pl.BlockSpec((_BM, D1), lambda i: (i, 0)),
            pl.BlockSpec((1, D1), lambda i: (0, 0)),
            pl.BlockSpec((D1, D1), lambda i: (0, 0)),
        ],
        out_specs=pl.BlockSpec((_BM, D1), lambda i: (i, 0)),
        out_shape=jax.ShapeDtypeStruct((NPAD, D1), jnp.float32),
    )(s1a, s1b, g1, dinvb, b1r, W2p)


_BM3 = 2000


def _mm3_body(s2a_ref, s2b_ref, g2_ref, dinvb_ref, b2_ref, o_ref):
    s = (s2a_ref[:] + s2b_ref[:] + g2_ref[:]) * dinvb_ref[:]
    o_ref[:] = s[:, :D2] + b2_ref[:]


def _mm3(s2a, s2b, g2, dinvb, b2r):
    return pl.pallas_call(
        _mm3_body,
        grid=(N // _BM3,),
        in_specs=[
            pl.BlockSpec((_BM3, D1), lambda i: (i, 0)),
            pl.BlockSpec((_BM3, D1), lambda i: (i, 0)),
            pl.BlockSpec((_BM3, D1), lambda i: (i, 0)),
            pl.BlockSpec((_BM3, D1), lambda i: (i, 0)),
            pl.BlockSpec((1, D2), lambda i: (0, 0)),
        ],
        out_specs=pl.BlockSpec((_BM3, D2), lambda i: (i, 0)),
        out_shape=jax.ShapeDtypeStruct((N, D2), jnp.float32),
    )(s2a, s2b, g2, dinvb, b2r)


def kernel(x, edge_index, W1, b1, W2, b2):
    ei = edge_index.astype(jnp.int32)
    pad = jnp.full((E_PAD - E,), N, jnp.int32)
    rowp = jnp.concatenate([ei[0], pad]).reshape(NW, K, CH)
    colp = jnp.concatenate([ei[1], pad]).reshape(NW, K, CH)
    x_pad = jnp.pad(x, ((0, NPAD - N), (0, 0)))

    W2p = jnp.pad(W2, ((0, 0), (0, D1 - D2)))

    deg_parts = _deg_sc(colp)
    xw1 = _mm1(x_pad, W1)
    dinvb, g1 = _scale(deg_parts, xw1)
    s1 = _push128(g1, rowp, colp)
    g2 = _mm2(s1[0], s1[1], g1, dinvb, b1.reshape(1, D1), W2p)
    s2 = _push128(g2, rowp, colp)
    return _mm3(s2[0], s2[1], g2, dinvb, b2.reshape(1, D2))
